# Initial kernel scaffold; baseline (speedup 1.0000x reference)
import jax
import jax.numpy as jnp
from jax.experimental import pallas as pl


def kernel(user_ids, item_ids, user_table, item_table, W1, b1, W2, b2, W3, b3):
    raise NotImplementedError("write your pallas kernel here")


# speedup vs baseline: 1.8649x; 1.8649x over previous
import jax
import jax.numpy as jnp
from jax import lax
from jax.experimental import pallas as pl
from jax.experimental.pallas import tpu as pltpu
from jax.experimental.pallas import tpu_sc as plsc

_NUSERS = 162541
_NITEMS = 59047
_BATCH = 16384
_D = 64
_VRU = (_NUSERS * _D) // 128
_VRI = (_NITEMS * _D) // 128
_NC = 2
_NS = 16
_NW = _NC * _NS
_BPW = _BATCH // _NW
_CHUNK = 128
_NCHUNK = _BPW // _CHUNK
_L = 16

_MLP_BB = 2048


def _sc_gather_body(uid_hbm, iid_hbm, utab_hbm, itab_hbm,
                    uout_hbm, iout_hbm,
                    ids_v, idx_v, rows_v, sem):
    wid = lax.axis_index("s") * _NC + lax.axis_index("c")
    base = wid * _BPW

    def one_table(id_hbm, tab_hbm, out_hbm, vr):
        pltpu.sync_copy(id_hbm.at[pl.ds(base, _BPW)], ids_v)
        for c in range(_BPW // _L):
            sl = pl.ds(c * _L, _L)
            idx_v[sl] = jnp.minimum(
                lax.shift_right_logical(ids_v[sl], 1), vr - 1)
        copies = []
        for c in range(_NCHUNK):
            sl = pl.ds(c * _CHUNK, _CHUNK)
            copies.append(
                pltpu.async_copy(tab_hbm.at[idx_v.at[sl]], rows_v.at[sl],
                                 sem))
        for cp in copies:
            cp.wait()
        pltpu.sync_copy(rows_v, out_hbm.at[pl.ds(base, _BPW)])

    one_table(uid_hbm, utab_hbm, uout_hbm, _VRU)
    one_table(iid_hbm, itab_hbm, iout_hbm, _VRI)


_SC_GATHER_CACHE = []


def _get_sc_gather():
    if not _SC_GATHER_CACHE:
        _SC_GATHER_CACHE.append(pl.kernel(
            _sc_gather_body,
            mesh=plsc.VectorSubcoreMesh(core_axis_name="c",
                                        subcore_axis_name="s"),
            out_type=[
                jax.ShapeDtypeStruct((_BATCH, 128), jnp.float32),
                jax.ShapeDtypeStruct((_BATCH, 128), jnp.float32),
            ],
            scratch_types=[
                pltpu.VMEM((_BPW,), jnp.int32),
                pltpu.VMEM((_BPW,), jnp.int32),
                pltpu.VMEM((_BPW, 128), jnp.float32),
                pltpu.SemaphoreType.DMA,
            ],
            compiler_params=pltpu.CompilerParams(use_tc_tiling_on_sc=False),
        ))
    return _SC_GATHER_CACHE[0]


def _mlp_body(u_ref, i_ref, uid_ref, iid_ref, ulast_ref, ilast_ref,
              w1_ref, b1_ref, w2_ref, b2_ref, w3_ref, b3_ref, o_ref):
    up = u_ref[...]
    ip = i_ref[...]
    uid = uid_ref[...]
    iid = iid_ref[...]
    u = jnp.where(lax.rem(uid, 2) == 1, up[:, _D:], up[:, :_D])
    u = jnp.where(uid == _NUSERS - 1, ulast_ref[...], u)
    v = jnp.where(lax.rem(iid, 2) == 1, ip[:, _D:], ip[:, :_D])
    v = jnp.where(iid == _NITEMS - 1, ilast_ref[...], v)
    w1 = w1_ref[...]
    h = lax.dot_general(u, w1[:, :_D], (((1,), (1,)), ((), ())),
                        preferred_element_type=jnp.float32)
    h = h + lax.dot_general(v, w1[:, _D:], (((1,), (1,)), ((), ())),
                            preferred_element_type=jnp.float32)
    h = jnp.maximum(h + b1_ref[...], 0.0)
    h = lax.dot_general(h, w2_ref[...], (((1,), (1,)), ((), ())),
                        preferred_element_type=jnp.float32)
    h = jnp.maximum(h + b2_ref[...], 0.0)
    o = lax.dot_general(h, w3_ref[...], (((1,), (1,)), ((), ())),
                        preferred_element_type=jnp.float32)
    o_ref[...] = o[:, :1] + b3_ref[0]


def _mlp(u_rows, i_rows, uid2d, iid2d, ulast, ilast, W1, b1, W2, b2, W3, b3):
    grid = (_BATCH // _MLP_BB,)
    return pl.pallas_call(
        _mlp_body,
        grid=grid,
        in_specs=[
            pl.BlockSpec((_MLP_BB, 128), lambda i: (i, 0)),
            pl.BlockSpec((_MLP_BB, 128), lambda i: (i, 0)),
            pl.BlockSpec((_MLP_BB, 1), lambda i: (i, 0)),
            pl.BlockSpec((_MLP_BB, 1), lambda i: (i, 0)),
            pl.BlockSpec((1, _D), lambda i: (0, 0)),
            pl.BlockSpec((1, _D), lambda i: (0, 0)),
            pl.BlockSpec((128, 2 * _D), lambda i: (0, 0)),
            pl.BlockSpec((1, 128), lambda i: (0, 0)),
            pl.BlockSpec((_D, 128), lambda i: (0, 0)),
            pl.BlockSpec((1, _D), lambda i: (0, 0)),
            pl.BlockSpec((128, _D), lambda i: (0, 0)),
            pl.BlockSpec(memory_space=pltpu.SMEM),
        ],
        out_specs=pl.BlockSpec((_MLP_BB, 1), lambda i: (i, 0)),
        out_shape=jax.ShapeDtypeStruct((_BATCH, 1), jnp.float32),
    )(u_rows, i_rows, uid2d, iid2d, ulast, ilast, W1, b1.reshape(1, -1), W2,
      b2.reshape(1, -1), jnp.pad(W3, ((0, 127), (0, 0))), b3)


def kernel(user_ids, item_ids, user_table, item_table, W1, b1, W2, b2, W3, b3):
    uids = user_ids.astype(jnp.int32)
    iids = item_ids.astype(jnp.int32)
    upairs = user_table.reshape(-1)[:_VRU * 128].reshape(_VRU, 128)
    ipairs = item_table.reshape(-1)[:_VRI * 128].reshape(_VRI, 128)
    u_rows, i_rows = _get_sc_gather()(uids, iids, upairs, ipairs)
    out2d = _mlp(u_rows, i_rows, uids.reshape(-1, 1), iids.reshape(-1, 1),
                 user_table[_NUSERS - 1:], item_table[_NITEMS - 1:],
                 W1, b1, W2, b2, W3, b3)
    return out2d[:, 0]


# speedup vs baseline: 2.1554x; 1.1557x over previous
import jax
import jax.numpy as jnp
from jax import lax
from jax.experimental import pallas as pl
from jax.experimental.pallas import tpu as pltpu
from jax.experimental.pallas import tpu_sc as plsc

_BATCH = 16384
_D = 64
_NC = 2
_NS = 16
_NW = _NC * _NS
_BPW = _BATCH // _NW
_CHUNK = 128
_NCHUNK = _BPW // _CHUNK

_MLP_BB = 2048


def _sc_gather_body(uid_hbm, iid_hbm, utab_hbm, itab_hbm, x_hbm,
                    uidx_v, iidx_v, urows_v, irows_v, sem):
    wid = lax.axis_index("s") * _NC + lax.axis_index("c")
    base = wid * _BPW
    pltpu.sync_copy(uid_hbm.at[pl.ds(base, _BPW)], uidx_v)
    pltpu.sync_copy(iid_hbm.at[pl.ds(base, _BPW)], iidx_v)
    copies = []
    for c in range(_NCHUNK):
        sl = pl.ds(c * _CHUNK, _CHUNK)
        copies.append(
            pltpu.async_copy(utab_hbm.at[uidx_v.at[sl]], urows_v.at[sl], sem))
        copies.append(
            pltpu.async_copy(itab_hbm.at[iidx_v.at[sl]], irows_v.at[sl], sem))
    for cp in copies:
        cp.wait()
    pltpu.sync_copy(urows_v, x_hbm.at[pl.ds(base, _BPW), pl.ds(0, _D)])
    pltpu.sync_copy(irows_v, x_hbm.at[pl.ds(base, _BPW), pl.ds(_D, _D)])


_SC_GATHER_CACHE = []


def _get_sc_gather():
    if not _SC_GATHER_CACHE:
        _SC_GATHER_CACHE.append(pl.kernel(
            _sc_gather_body,
            mesh=plsc.VectorSubcoreMesh(core_axis_name="c",
                                        subcore_axis_name="s"),
            out_type=jax.ShapeDtypeStruct((_BATCH, 2 * _D), jnp.float32),
            scratch_types=[
                pltpu.VMEM((_BPW,), jnp.int32),
                pltpu.VMEM((_BPW,), jnp.int32),
                pltpu.VMEM((_BPW, _D), jnp.float32),
                pltpu.VMEM((_BPW, _D), jnp.float32),
                pltpu.SemaphoreType.DMA,
            ],
            compiler_params=pltpu.CompilerParams(use_tc_tiling_on_sc=False),
        ))
    return _SC_GATHER_CACHE[0]


def _mlp_body(x_ref, w1_ref, w2_ref, w3_ref, o_ref):
    ones = jnp.full((_MLP_BB, 8), 1.0, jnp.float32)
    xa = jnp.concatenate([x_ref[...], ones], axis=1)
    h = lax.dot_general(xa, w1_ref[...], (((1,), (1,)), ((), ())),
                        preferred_element_type=jnp.float32)
    h = jnp.maximum(h, 0.0)
    ha = jnp.concatenate([h, ones], axis=1)
    h2 = lax.dot_general(ha, w2_ref[...], (((1,), (1,)), ((), ())),
                         preferred_element_type=jnp.float32)
    h2 = jnp.maximum(h2, 0.0)
    h2a = jnp.concatenate([h2, ones], axis=1)
    o_ref[...] = lax.dot_general(h2a, w3_ref[...], (((1,), (1,)), ((), ())),
                                 preferred_element_type=jnp.float32)


def _mlp(x, W1a, W2a, W3a):
    grid = (_BATCH // _MLP_BB,)
    return pl.pallas_call(
        _mlp_body,
        grid=grid,
        in_specs=[
            pl.BlockSpec((_MLP_BB, 2 * _D), lambda i: (i, 0)),
            pl.BlockSpec((128, 136), lambda i: (0, 0)),
            pl.BlockSpec((_D, 136), lambda i: (0, 0)),
            pl.BlockSpec((8, 72), lambda i: (0, 0)),
        ],
        out_specs=pl.BlockSpec((_MLP_BB, 8), lambda i: (i, 0)),
        out_shape=jax.ShapeDtypeStruct((_BATCH, 8), jnp.float32),
    )(x, W1a, W2a, W3a)


def kernel(user_ids, item_ids, user_table, item_table, W1, b1, W2, b2, W3, b3):
    uids = user_ids.astype(jnp.int32)
    iids = item_ids.astype(jnp.int32)
    x = _get_sc_gather()(uids, iids, user_table, item_table)
    W1a = jnp.concatenate(
        [W1, b1.reshape(-1, 1), jnp.zeros((128, 7), jnp.float32)], axis=1)
    W2a = jnp.concatenate(
        [W2, b2.reshape(-1, 1), jnp.zeros((_D, 7), jnp.float32)], axis=1)
    W3a = jnp.pad(
        jnp.concatenate(
            [W3, b3.reshape(-1, 1), jnp.zeros((1, 7), jnp.float32)], axis=1),
        ((0, 7), (0, 0)))
    out8 = _mlp(x, W1a, W2a, W3a)
    return out8[:, 0]


# speedup vs baseline: 2.2806x; 1.0581x over previous
import jax
import jax.numpy as jnp
from jax import lax
from jax.experimental import pallas as pl
from jax.experimental.pallas import tpu as pltpu
from jax.experimental.pallas import tpu_sc as plsc

_BATCH = 16384
_D = 64
_NC = 2
_NS = 16
_NW = _NC * _NS
_BPW = _BATCH // _NW
_CHUNK = 128
_NCHUNK = _BPW // _CHUNK

_MLP_BB = 4096


def _sc_gather_body(uid_hbm, iid_hbm, utab_hbm, itab_hbm, x_hbm,
                    uidx_v, iidx_v, urows_v, irows_v, sem):
    wid = lax.axis_index("s") * _NC + lax.axis_index("c")
    base = wid * _BPW
    pltpu.sync_copy(uid_hbm.at[pl.ds(base, _BPW)], uidx_v)
    pltpu.sync_copy(iid_hbm.at[pl.ds(base, _BPW)], iidx_v)
    copies = []
    for c in range(_NCHUNK):
        sl = pl.ds(c * _CHUNK, _CHUNK)
        copies.append(
            pltpu.async_copy(utab_hbm.at[uidx_v.at[sl]], urows_v.at[sl], sem))
        copies.append(
            pltpu.async_copy(itab_hbm.at[iidx_v.at[sl]], irows_v.at[sl], sem))
    for cp in copies:
        cp.wait()
    pltpu.sync_copy(urows_v, x_hbm.at[pl.ds(base, _BPW), pl.ds(0, _D)])
    pltpu.sync_copy(irows_v, x_hbm.at[pl.ds(base, _BPW), pl.ds(_D, _D)])


_SC_GATHER_CACHE = []


def _get_sc_gather():
    if not _SC_GATHER_CACHE:
        _SC_GATHER_CACHE.append(pl.kernel(
            _sc_gather_body,
            mesh=plsc.VectorSubcoreMesh(core_axis_name="c",
                                        subcore_axis_name="s"),
            out_type=jax.ShapeDtypeStruct((_BATCH, 2 * _D), jnp.float32),
            scratch_types=[
                pltpu.VMEM((_BPW,), jnp.int32),
                pltpu.VMEM((_BPW,), jnp.int32),
                pltpu.VMEM((_BPW, _D), jnp.float32),
                pltpu.VMEM((_BPW, _D), jnp.float32),
                pltpu.SemaphoreType.DMA,
            ],
            compiler_params=pltpu.CompilerParams(use_tc_tiling_on_sc=False),
        ))
    return _SC_GATHER_CACHE[0]


def _mlp_body(x_ref, w1_ref, b1_ref, w2_ref, b2_ref, w3_ref, b3_ref, o_ref):
    x = x_ref[...]
    h = lax.dot_general(x, w1_ref[...], (((1,), (1,)), ((), ())),
                        preferred_element_type=jnp.float32)
    h = jnp.maximum(h + b1_ref[...], 0.0)
    h = lax.dot_general(h, w2_ref[...], (((1,), (1,)), ((), ())),
                        preferred_element_type=jnp.float32)
    h = jnp.maximum(h + b2_ref[...], 0.0)
    o = lax.dot_general(w3_ref[...], h, (((1,), (1,)), ((), ())),
                        preferred_element_type=jnp.float32)
    o_ref[...] = o + b3_ref[0]


def _mlp(x, W1, b1, W2, b2, W3p, b3):
    grid = (_BATCH // _MLP_BB,)
    return pl.pallas_call(
        _mlp_body,
        grid=grid,
        in_specs=[
            pl.BlockSpec((_MLP_BB, 2 * _D), lambda i: (i, 0)),
            pl.BlockSpec((128, 128), lambda i: (0, 0)),
            pl.BlockSpec((1, 128), lambda i: (0, 0)),
            pl.BlockSpec((_D, 128), lambda i: (0, 0)),
            pl.BlockSpec((1, _D), lambda i: (0, 0)),
            pl.BlockSpec((8, _D), lambda i: (0, 0)),
            pl.BlockSpec(memory_space=pltpu.SMEM),
        ],
        out_specs=pl.BlockSpec((8, _MLP_BB), lambda i: (0, i)),
        out_shape=jax.ShapeDtypeStruct((8, _BATCH), jnp.float32),
    )(x, W1, b1.reshape(1, -1), W2, b2.reshape(1, -1), W3p, b3)


def kernel(user_ids, item_ids, user_table, item_table, W1, b1, W2, b2, W3, b3):
    uids = user_ids.astype(jnp.int32)
    iids = item_ids.astype(jnp.int32)
    x = _get_sc_gather()(uids, iids, user_table, item_table)
    out8 = _mlp(x, W1, b1, W2, b2, jnp.pad(W3, ((0, 7), (0, 0))), b3)
    return out8[0]
